# baseline (speedup 1.0000x reference)
import functools

import jax
import jax.numpy as jnp
from jax import lax
from jax.experimental import pallas as pl
from jax.experimental.pallas import tpu as pltpu
from jax.experimental.pallas import tpu_sc as plsc

_C = 128
_H = 128
_W = 128
_HW = _H * _W
_HG = 7
_WG = 25281
_P = _HG * _WG
_B = 2048
_NCHUNK = -(-_P // _B)
_P_PAD = _NCHUNK * _B
_NTILE = 32
_CPT = _C // _NTILE
_NG = _B // 16
_UNROLL = 4


def _sc_grid_sample(planes, gx, gy):
  mesh = plsc.VectorSubcoreMesh(core_axis_name="c", subcore_axis_name="s")

  @functools.partial(
      pl.kernel,
      out_type=jax.ShapeDtypeStruct((_C, _P_PAD), jnp.float32),
      mesh=mesh,
      compiler_params=pltpu.CompilerParams(needs_layout_passes=False),
      scratch_types=[
          pltpu.VMEM((_CPT * _HW,), jnp.float32),
          pltpu.VMEM((_B,), jnp.float32),
          pltpu.VMEM((_B,), jnp.float32),
          pltpu.VMEM((_CPT, _B), jnp.float32),
      ],
  )
  def k(planes_hbm, gx_hbm, gy_hbm, out_hbm, plane_v, gx_v, gy_v, out_v):
    wid = lax.axis_index("c") * 16 + lax.axis_index("s")
    c0 = wid * _CPT
    pltpu.sync_copy(planes_hbm.at[pl.ds(c0 * _HW, _CPT * _HW)], plane_v)

    def chunk_body(ci, carry):
      base = ci * _B
      pltpu.sync_copy(gx_hbm.at[pl.ds(base, _B)], gx_v)
      pltpu.sync_copy(gy_hbm.at[pl.ds(base, _B)], gy_v)

      def group_body(g, gcarry):
       for u in range(_UNROLL):
        off = g * (16 * _UNROLL) + u * 16
        gx16 = gx_v[pl.ds(off, 16)]
        gy16 = gy_v[pl.ds(off, 16)]
        ix = ((gx16 + 1.0) * 128.0 - 1.0) * 0.5
        iy = ((gy16 + 1.0) * 128.0 - 1.0) * 0.5
        ix = jnp.minimum(jnp.maximum(ix, -2.0), 129.0)
        iy = jnp.minimum(jnp.maximum(iy, -2.0), 129.0)
        tx = ix.astype(jnp.int32).astype(jnp.float32)
        ty = iy.astype(jnp.int32).astype(jnp.float32)
        fx0 = jnp.where(tx > ix, tx - 1.0, tx)
        fy0 = jnp.where(ty > iy, ty - 1.0, ty)
        fx1 = fx0 + 1.0
        fy1 = fy0 + 1.0
        wx1 = ix - fx0
        wx0 = 1.0 - wx1
        wy1 = iy - fy0
        wy0 = 1.0 - wy1
        vx0 = (fx0 >= 0.0) & (fx0 <= 127.0)
        vx1 = (fx1 >= 0.0) & (fx1 <= 127.0)
        vy0 = (fy0 >= 0.0) & (fy0 <= 127.0)
        vy1 = (fy1 >= 0.0) & (fy1 <= 127.0)
        zero = jnp.zeros((16,), jnp.float32)
        w00 = jnp.where(vx0 & vy0, wx0 * wy0, zero)
        w01 = jnp.where(vx1 & vy0, wx1 * wy0, zero)
        w10 = jnp.where(vx0 & vy1, wx0 * wy1, zero)
        w11 = jnp.where(vx1 & vy1, wx1 * wy1, zero)
        x0 = jnp.minimum(jnp.maximum(fx0, 0.0), 127.0).astype(jnp.int32)
        x1 = jnp.minimum(jnp.maximum(fx1, 0.0), 127.0).astype(jnp.int32)
        y0 = jnp.minimum(jnp.maximum(fy0, 0.0), 127.0).astype(jnp.int32)
        y1 = jnp.minimum(jnp.maximum(fy1, 0.0), 127.0).astype(jnp.int32)
        i00 = y0 * _W + x0
        i01 = y0 * _W + x1
        i10 = y1 * _W + x0
        i11 = y1 * _W + x1
        for c in range(_CPT):
          cb = jnp.full((16,), c * _HW, jnp.int32)
          v00 = plsc.load_gather(plane_v, [cb + i00])
          v01 = plsc.load_gather(plane_v, [cb + i01])
          v10 = plsc.load_gather(plane_v, [cb + i10])
          v11 = plsc.load_gather(plane_v, [cb + i11])
          acc = v00 * w00 + v01 * w01 + v10 * w10 + v11 * w11
          out_v[c, pl.ds(off, 16)] = acc
       return gcarry

      lax.fori_loop(0, _NG // _UNROLL, group_body, 0)
      for c in range(_CPT):
        pltpu.sync_copy(out_v.at[c], out_hbm.at[c0 + c, pl.ds(base, _B)])
      return carry

    lax.fori_loop(0, _NCHUNK, chunk_body, 0)

  return k(planes, gx, gy)


def kernel(input_tensor, grid):
  planes = input_tensor.reshape(_C * _HW)
  g = grid.reshape(_P, 2)
  gx = jnp.pad(g[:, 0], (0, _P_PAD - _P))
  gy = jnp.pad(g[:, 1], (0, _P_PAD - _P))
  out = _sc_grid_sample(planes, gx, gy)
  return out[:, :_P].reshape(1, _C, _HG, _WG)


# speedup vs baseline: 1.0519x; 1.0519x over previous
import functools

import jax
import jax.numpy as jnp
from jax import lax
from jax.experimental import pallas as pl
from jax.experimental.pallas import tpu as pltpu
from jax.experimental.pallas import tpu_sc as plsc

_C = 128
_H = 128
_W = 128
_HW = _H * _W
_HG = 7
_WG = 25281
_P = _HG * _WG
_B = 8192
_NCHUNK = -(-_P // _B)
_P_PAD = _NCHUNK * _B
_NTILE = 32
_CPT = _C // _NTILE
_NG = _B // 16
_UNROLL = 4


def _sc_grid_sample(planes, gx, gy):
  mesh = plsc.VectorSubcoreMesh(core_axis_name="c", subcore_axis_name="s")

  @functools.partial(
      pl.kernel,
      out_type=jax.ShapeDtypeStruct((_C, _P_PAD), jnp.float32),
      mesh=mesh,
      compiler_params=pltpu.CompilerParams(needs_layout_passes=False),
      scratch_types=[
          pltpu.VMEM((_CPT * _HW,), jnp.float32),
          pltpu.VMEM((_B,), jnp.float32),
          pltpu.VMEM((_B,), jnp.float32),
          pltpu.VMEM((_CPT, _B), jnp.float32),
      ],
  )
  def k(planes_hbm, gx_hbm, gy_hbm, out_hbm, plane_v, gx_v, gy_v, out_v):
    wid = lax.axis_index("c") * 16 + lax.axis_index("s")
    c0 = wid * _CPT
    pltpu.sync_copy(planes_hbm.at[pl.ds(c0 * _HW, _CPT * _HW)], plane_v)

    def chunk_body(ci, carry):
      base = ci * _B
      pltpu.sync_copy(gx_hbm.at[pl.ds(base, _B)], gx_v)
      pltpu.sync_copy(gy_hbm.at[pl.ds(base, _B)], gy_v)

      def group_body(g, gcarry):
       for u in range(_UNROLL):
        off = g * (16 * _UNROLL) + u * 16
        gx16 = gx_v[pl.ds(off, 16)]
        gy16 = gy_v[pl.ds(off, 16)]
        ix = ((gx16 + 1.0) * 128.0 - 1.0) * 0.5
        iy = ((gy16 + 1.0) * 128.0 - 1.0) * 0.5
        ix = jnp.minimum(jnp.maximum(ix, -2.0), 129.0)
        iy = jnp.minimum(jnp.maximum(iy, -2.0), 129.0)
        tx = ix.astype(jnp.int32).astype(jnp.float32)
        ty = iy.astype(jnp.int32).astype(jnp.float32)
        fx0 = jnp.where(tx > ix, tx - 1.0, tx)
        fy0 = jnp.where(ty > iy, ty - 1.0, ty)
        fx1 = fx0 + 1.0
        fy1 = fy0 + 1.0
        wx1 = ix - fx0
        wx0 = 1.0 - wx1
        wy1 = iy - fy0
        wy0 = 1.0 - wy1
        vx0 = (fx0 >= 0.0) & (fx0 <= 127.0)
        vx1 = (fx1 >= 0.0) & (fx1 <= 127.0)
        vy0 = (fy0 >= 0.0) & (fy0 <= 127.0)
        vy1 = (fy1 >= 0.0) & (fy1 <= 127.0)
        zero = jnp.zeros((16,), jnp.float32)
        w00 = jnp.where(vx0 & vy0, wx0 * wy0, zero)
        w01 = jnp.where(vx1 & vy0, wx1 * wy0, zero)
        w10 = jnp.where(vx0 & vy1, wx0 * wy1, zero)
        w11 = jnp.where(vx1 & vy1, wx1 * wy1, zero)
        x0 = jnp.minimum(jnp.maximum(fx0, 0.0), 127.0).astype(jnp.int32)
        x1 = jnp.minimum(jnp.maximum(fx1, 0.0), 127.0).astype(jnp.int32)
        y0 = jnp.minimum(jnp.maximum(fy0, 0.0), 127.0).astype(jnp.int32)
        y1 = jnp.minimum(jnp.maximum(fy1, 0.0), 127.0).astype(jnp.int32)
        i00 = y0 * _W + x0
        i01 = y0 * _W + x1
        i10 = y1 * _W + x0
        i11 = y1 * _W + x1
        for c in range(_CPT):
          cb = jnp.full((16,), c * _HW, jnp.int32)
          v00 = plsc.load_gather(plane_v, [cb + i00])
          v01 = plsc.load_gather(plane_v, [cb + i01])
          v10 = plsc.load_gather(plane_v, [cb + i10])
          v11 = plsc.load_gather(plane_v, [cb + i11])
          acc = v00 * w00 + v01 * w01 + v10 * w10 + v11 * w11
          out_v[c, pl.ds(off, 16)] = acc
       return gcarry

      lax.fori_loop(0, _NG // _UNROLL, group_body, 0)
      pltpu.sync_copy(out_v, out_hbm.at[pl.ds(c0, _CPT), pl.ds(base, _B)])
      return carry

    lax.fori_loop(0, _NCHUNK, chunk_body, 0)

  return k(planes, gx, gy)


def kernel(input_tensor, grid):
  planes = input_tensor.reshape(_C * _HW)
  g = grid.reshape(_P, 2)
  gx = jnp.pad(g[:, 0], (0, _P_PAD - _P))
  gy = jnp.pad(g[:, 1], (0, _P_PAD - _P))
  out = _sc_grid_sample(planes, gx, gy)
  return out[:, :_P].reshape(1, _C, _HG, _WG)


# speedup vs baseline: 1.1658x; 1.1083x over previous
import functools

import jax
import jax.numpy as jnp
from jax import lax
from jax.experimental import pallas as pl
from jax.experimental.pallas import tpu as pltpu
from jax.experimental.pallas import tpu_sc as plsc

_C = 128
_H = 128
_W = 128
_HW = _H * _W
_HG = 7
_WG = 25281
_P = _HG * _WG
_B = 8192
_NCHUNK = -(-_P // _B)
_P_PAD = _NCHUNK * _B
_NTILE = 32
_CPT = _C // _NTILE
_NG = _B // 16
_UNROLL = 4


def _sc_grid_sample(planes, gx, gy):
  mesh = plsc.VectorSubcoreMesh(core_axis_name="c", subcore_axis_name="s")

  @functools.partial(
      pl.kernel,
      out_type=jax.ShapeDtypeStruct((_C, _P_PAD), jnp.float32),
      mesh=mesh,
      compiler_params=pltpu.CompilerParams(needs_layout_passes=False),
      scratch_types=[
          pltpu.VMEM((_CPT * _HW,), jnp.float32),
          pltpu.VMEM((_B,), jnp.float32),
          pltpu.VMEM((_B,), jnp.float32),
          pltpu.VMEM((_CPT, _B), jnp.float32),
      ],
  )
  def k(planes_hbm, gx_hbm, gy_hbm, out_hbm, plane_v, gx_v, gy_v, out_v):
    wid = lax.axis_index("c") * 16 + lax.axis_index("s")
    c0 = wid * _CPT
    pltpu.sync_copy(planes_hbm.at[pl.ds(c0 * _HW, _CPT * _HW)], plane_v)

    def chunk_body(ci, carry):
      base = ci * _B
      pltpu.sync_copy(gx_hbm.at[pl.ds(base, _B)], gx_v)
      pltpu.sync_copy(gy_hbm.at[pl.ds(base, _B)], gy_v)

      @plsc.parallel_loop(0, _NG, step=1, unroll=_UNROLL)
      def group_body(g):
        off = g * 16
        gx16 = gx_v[pl.ds(off, 16)]
        gy16 = gy_v[pl.ds(off, 16)]
        ix = ((gx16 + 1.0) * 128.0 - 1.0) * 0.5
        iy = ((gy16 + 1.0) * 128.0 - 1.0) * 0.5
        ix = jnp.minimum(jnp.maximum(ix, -2.0), 129.0)
        iy = jnp.minimum(jnp.maximum(iy, -2.0), 129.0)
        tx = ix.astype(jnp.int32).astype(jnp.float32)
        ty = iy.astype(jnp.int32).astype(jnp.float32)
        fx0 = jnp.where(tx > ix, tx - 1.0, tx)
        fy0 = jnp.where(ty > iy, ty - 1.0, ty)
        fx1 = fx0 + 1.0
        fy1 = fy0 + 1.0
        wx1 = ix - fx0
        wx0 = 1.0 - wx1
        wy1 = iy - fy0
        wy0 = 1.0 - wy1
        vx0 = (fx0 >= 0.0) & (fx0 <= 127.0)
        vx1 = (fx1 >= 0.0) & (fx1 <= 127.0)
        vy0 = (fy0 >= 0.0) & (fy0 <= 127.0)
        vy1 = (fy1 >= 0.0) & (fy1 <= 127.0)
        zero = jnp.zeros((16,), jnp.float32)
        w00 = jnp.where(vx0 & vy0, wx0 * wy0, zero)
        w01 = jnp.where(vx1 & vy0, wx1 * wy0, zero)
        w10 = jnp.where(vx0 & vy1, wx0 * wy1, zero)
        w11 = jnp.where(vx1 & vy1, wx1 * wy1, zero)
        x0 = jnp.minimum(jnp.maximum(fx0, 0.0), 127.0).astype(jnp.int32)
        x1 = jnp.minimum(jnp.maximum(fx1, 0.0), 127.0).astype(jnp.int32)
        y0 = jnp.minimum(jnp.maximum(fy0, 0.0), 127.0).astype(jnp.int32)
        y1 = jnp.minimum(jnp.maximum(fy1, 0.0), 127.0).astype(jnp.int32)
        i00 = y0 * _W + x0
        i01 = y0 * _W + x1
        i10 = y1 * _W + x0
        i11 = y1 * _W + x1
        for c in range(_CPT):
          cb = jnp.full((16,), c * _HW, jnp.int32)
          v00 = plsc.load_gather(plane_v, [cb + i00])
          v01 = plsc.load_gather(plane_v, [cb + i01])
          v10 = plsc.load_gather(plane_v, [cb + i10])
          v11 = plsc.load_gather(plane_v, [cb + i11])
          acc = v00 * w00 + v01 * w01 + v10 * w10 + v11 * w11
          out_v[c, pl.ds(off, 16)] = acc

      pltpu.sync_copy(out_v, out_hbm.at[pl.ds(c0, _CPT), pl.ds(base, _B)])
      return carry

    lax.fori_loop(0, _NCHUNK, chunk_body, 0)

  return k(planes, gx, gy)


def kernel(input_tensor, grid):
  planes = input_tensor.reshape(_C * _HW)
  g = grid.reshape(_P, 2)
  gx = jnp.pad(g[:, 0], (0, _P_PAD - _P))
  gy = jnp.pad(g[:, 1], (0, _P_PAD - _P))
  out = _sc_grid_sample(planes, gx, gy)
  return out[:, :_P].reshape(1, _C, _HG, _WG)


# speedup vs baseline: 1.6606x; 1.4245x over previous
import functools

import jax
import jax.numpy as jnp
from jax import lax
from jax.experimental import pallas as pl
from jax.experimental.pallas import tpu as pltpu
from jax.experimental.pallas import tpu_sc as plsc

_C = 128
_H = 128
_W = 128
_HG = 7
_WG = 25281
_P = _HG * _WG
_PR = _H + 3
_PSTR = _W + 4
_PL = _PR * _PSTR
_B = 8192
_NFULL = _P // _B
_TAILW = 5120
_TBASE = _NFULL * _B
_TAILG = _TAILW // 16
_P_PAD = _TBASE + _TAILW
_NTILE = 32
_CPT = _C // _NTILE
_NG = _B // 16
_UNROLL = 4


def _sc_grid_sample(planes, gx, gy):
  mesh = plsc.VectorSubcoreMesh(core_axis_name="c", subcore_axis_name="s")

  @functools.partial(
      pl.kernel,
      out_type=jax.ShapeDtypeStruct((_C, _P_PAD), jnp.float32),
      mesh=mesh,
      compiler_params=pltpu.CompilerParams(needs_layout_passes=False),
      scratch_types=[
          pltpu.VMEM((_PL,), jnp.float32),
          pltpu.VMEM((_PL,), jnp.float32),
          pltpu.VMEM((_PL,), jnp.float32),
          pltpu.VMEM((_PL,), jnp.float32),
          pltpu.VMEM((_B,), jnp.float32),
          pltpu.VMEM((_B,), jnp.float32),
          pltpu.VMEM((_CPT, _B), jnp.float32),
      ],
  )
  def k(planes_hbm, gx_hbm, gy_hbm, out_hbm, p0, p1, p2, p3, gx_v, gy_v,
        out_v):
    wid = lax.axis_index("c") * 16 + lax.axis_index("s")
    c0 = wid * _CPT
    pltpu.sync_copy(planes_hbm.at[c0], p0)
    pltpu.sync_copy(planes_hbm.at[c0 + 1], p1)
    pltpu.sync_copy(planes_hbm.at[c0 + 2], p2)
    pltpu.sync_copy(planes_hbm.at[c0 + 3], p3)

    def do_groups(ngroups, lane_off=0):
      @plsc.parallel_loop(0, ngroups, step=1, unroll=_UNROLL)
      def group_body(g):
        off = g * 16
        if lane_off == 0:
          gx16 = gx_v[pl.ds(off, 16)]
          gy16 = gy_v[pl.ds(off, 16)]
        else:
          pos = lax.iota(jnp.int32, 16) + (off + lane_off)
          gx16 = plsc.load_gather(gx_v, [pos])
          gy16 = plsc.load_gather(gy_v, [pos])
        ix = ((gx16 + 1.0) * 128.0 - 1.0) * 0.5
        iy = ((gy16 + 1.0) * 128.0 - 1.0) * 0.5
        ix = jnp.minimum(jnp.maximum(ix, -1.0), 128.0)
        iy = jnp.minimum(jnp.maximum(iy, -1.0), 128.0)
        bx = ix + 1.0
        by = iy + 1.0
        px0 = bx.astype(jnp.int32)
        py0 = by.astype(jnp.int32)
        wx1 = bx - px0.astype(jnp.float32)
        wy1 = by - py0.astype(jnp.float32)
        wx0 = 1.0 - wx1
        wy0 = 1.0 - wy1
        rb = (py0 << 7) + (py0 << 2) + px0
        i01 = rb + 1
        i10 = rb + _PSTR
        i11 = rb + (_PSTR + 1)
        w00 = wx0 * wy0
        w01 = wx1 * wy0
        w10 = wx0 * wy1
        w11 = wx1 * wy1
        for c, pv in enumerate((p0, p1, p2, p3)):
          v00 = plsc.load_gather(pv, [rb])
          v01 = plsc.load_gather(pv, [i01])
          v10 = plsc.load_gather(pv, [i10])
          v11 = plsc.load_gather(pv, [i11])
          acc = v00 * w00 + v01 * w01 + v10 * w10 + v11 * w11
          out_v[c, pl.ds(off, 16)] = acc

    def chunk_body(ci, carry):
      base = ci * _B
      pltpu.sync_copy(gx_hbm.at[pl.ds(base, _B)], gx_v)
      pltpu.sync_copy(gy_hbm.at[pl.ds(base, _B)], gy_v)
      do_groups(_NG)
      pltpu.sync_copy(out_v, out_hbm.at[pl.ds(c0, _CPT), pl.ds(base, _B)])
      return carry

    lax.fori_loop(0, _NFULL, chunk_body, 0)

    pltpu.sync_copy(gx_hbm.at[pl.ds(_TBASE, _TAILW)],
                    gx_v.at[pl.ds(0, _TAILW)])
    pltpu.sync_copy(gy_hbm.at[pl.ds(_TBASE, _TAILW)],
                    gy_v.at[pl.ds(0, _TAILW)])
    do_groups(_TAILG)
    pltpu.sync_copy(out_v.at[pl.ds(0, _CPT), pl.ds(0, _TAILW)],
                    out_hbm.at[pl.ds(c0, _CPT), pl.ds(_TBASE, _TAILW)])

  return k(planes, gx, gy)


def kernel(input_tensor, grid):
  planes = jnp.pad(input_tensor[0], ((0, 0), (1, 2), (1, 3)))
  planes = planes.reshape(_C, _PL)
  g = grid.reshape(_P, 2)
  gx = jnp.pad(g[:, 0], (0, _P_PAD - _P))
  gy = jnp.pad(g[:, 1], (0, _P_PAD - _P))
  out = _sc_grid_sample(planes, gx, gy)
  return out[:, :_P].reshape(1, _C, _HG, _WG)


# speedup vs baseline: 1.8745x; 1.1288x over previous
import functools

import jax
import jax.numpy as jnp
from jax import lax
from jax.experimental import pallas as pl
from jax.experimental.pallas import tpu as pltpu
from jax.experimental.pallas import tpu_sc as plsc

_C = 128
_H = 128
_W = 128
_HG = 7
_WG = 25281
_P = _HG * _WG
_PR = _H + 3
_PSTR = _W + 4
_PL = _PR * _PSTR
_B = 4096
_NFULL = _P // _B
_TAILW = 1024
_TBASE = _NFULL * _B
_P_PAD = _TBASE + _TAILW
_NPAIR = 21
_NTILE = 32
_CPT = _C // _NTILE
_NG = _B // 16
_UNROLL = 4


def _sc_grid_sample(planes, gx, gy):
  mesh = plsc.VectorSubcoreMesh(core_axis_name="c", subcore_axis_name="s")

  @functools.partial(
      pl.kernel,
      out_type=jax.ShapeDtypeStruct((_C, _P_PAD), jnp.float32),
      mesh=mesh,
      compiler_params=pltpu.CompilerParams(needs_layout_passes=False),
      scratch_types=[
          pltpu.VMEM((_PL,), jnp.float32),
          pltpu.VMEM((_PL,), jnp.float32),
          pltpu.VMEM((_PL,), jnp.float32),
          pltpu.VMEM((_PL,), jnp.float32),
          pltpu.VMEM((_B,), jnp.float32),
          pltpu.VMEM((_B,), jnp.float32),
          pltpu.VMEM((_B,), jnp.float32),
          pltpu.VMEM((_B,), jnp.float32),
          pltpu.VMEM((_CPT, _B), jnp.float32),
          pltpu.VMEM((_CPT, _B), jnp.float32),
          pltpu.SemaphoreType.DMA,
          pltpu.SemaphoreType.DMA,
          pltpu.SemaphoreType.DMA,
          pltpu.SemaphoreType.DMA,
      ],
  )
  def k(planes_hbm, gx_hbm, gy_hbm, out_hbm, p0, p1, p2, p3,
        gx0, gy0, gx1, gy1, out0, out1, in0, in1, st0, st1):
    wid = lax.axis_index("c") * 16 + lax.axis_index("s")
    c0 = wid * _CPT
    pltpu.sync_copy(planes_hbm.at[c0], p0)
    pltpu.sync_copy(planes_hbm.at[c0 + 1], p1)
    pltpu.sync_copy(planes_hbm.at[c0 + 2], p2)
    pltpu.sync_copy(planes_hbm.at[c0 + 3], p3)

    def start_in(base, gxb, gyb, sem):
      pltpu.async_copy(gx_hbm.at[pl.ds(base, _B)], gxb, sem)
      pltpu.async_copy(gy_hbm.at[pl.ds(base, _B)], gyb, sem)

    def wait_in(gxb, gyb, sem):
      pltpu.make_async_copy(gx_hbm.at[pl.ds(0, _B)], gxb, sem).wait()
      pltpu.make_async_copy(gy_hbm.at[pl.ds(0, _B)], gyb, sem).wait()

    def start_store(base, outb, sem):
      pltpu.async_copy(outb, out_hbm.at[pl.ds(c0, _CPT), pl.ds(base, _B)],
                       sem)

    def wait_store(outb, sem):
      pltpu.make_async_copy(
          outb, out_hbm.at[pl.ds(c0, _CPT), pl.ds(0, _B)], sem).wait()

    def compute(gxb, gyb, outb, ngroups):
      @plsc.parallel_loop(0, ngroups, step=1, unroll=_UNROLL)
      def group_body(g):
        off = g * 16
        gx16 = gxb[pl.ds(off, 16)]
        gy16 = gyb[pl.ds(off, 16)]
        ix = ((gx16 + 1.0) * 128.0 - 1.0) * 0.5
        iy = ((gy16 + 1.0) * 128.0 - 1.0) * 0.5
        ix = jnp.minimum(jnp.maximum(ix, -1.0), 128.0)
        iy = jnp.minimum(jnp.maximum(iy, -1.0), 128.0)
        bx = ix + 1.0
        by = iy + 1.0
        px0 = bx.astype(jnp.int32)
        py0 = by.astype(jnp.int32)
        wx1 = bx - px0.astype(jnp.float32)
        wy1 = by - py0.astype(jnp.float32)
        wx0 = 1.0 - wx1
        wy0 = 1.0 - wy1
        rb = (py0 << 7) + (py0 << 2) + px0
        i01 = rb + 1
        i10 = rb + _PSTR
        i11 = rb + (_PSTR + 1)
        w00 = wx0 * wy0
        w01 = wx1 * wy0
        w10 = wx0 * wy1
        w11 = wx1 * wy1
        for c, pv in enumerate((p0, p1, p2, p3)):
          v00 = plsc.load_gather(pv, [rb])
          v01 = plsc.load_gather(pv, [i01])
          v10 = plsc.load_gather(pv, [i10])
          v11 = plsc.load_gather(pv, [i11])
          acc = v00 * w00 + v01 * w01 + v10 * w10 + v11 * w11
          outb[c, pl.ds(off, 16)] = acc

    start_in(0, gx0, gy0, in0)

    def pair_body(i, carry):
      start_in((2 * i + 1) * _B, gx1, gy1, in1)
      wait_in(gx0, gy0, in0)

      @pl.when(i > 0)
      def _():
        wait_store(out0, st0)

      compute(gx0, gy0, out0, _NG)
      start_store(2 * i * _B, out0, st0)
      start_in((2 * i + 2) * _B, gx0, gy0, in0)
      wait_in(gx1, gy1, in1)

      @pl.when(i > 0)
      def _():
        wait_store(out1, st1)

      compute(gx1, gy1, out1, _NG)
      start_store((2 * i + 1) * _B, out1, st1)
      return carry

    lax.fori_loop(0, _NPAIR, pair_body, 0)

    pltpu.async_copy(gx_hbm.at[pl.ds(_TBASE, _TAILW)],
                     gx1.at[pl.ds(0, _TAILW)], in1)
    pltpu.async_copy(gy_hbm.at[pl.ds(_TBASE, _TAILW)],
                     gy1.at[pl.ds(0, _TAILW)], in1)
    wait_in(gx0, gy0, in0)
    wait_store(out0, st0)
    compute(gx0, gy0, out0, _NG)
    start_store(42 * _B, out0, st0)
    pltpu.make_async_copy(gx_hbm.at[pl.ds(0, _TAILW)],
                          gx1.at[pl.ds(0, _TAILW)], in1).wait()
    pltpu.make_async_copy(gy_hbm.at[pl.ds(0, _TAILW)],
                          gy1.at[pl.ds(0, _TAILW)], in1).wait()
    wait_store(out1, st1)
    compute(gx1, gy1, out1, _TAILW // 16)
    pltpu.async_copy(out1.at[pl.ds(0, _CPT), pl.ds(0, _TAILW)],
                     out_hbm.at[pl.ds(c0, _CPT), pl.ds(_TBASE, _TAILW)],
                     st1)
    wait_store(out0, st0)
    pltpu.make_async_copy(
        out1.at[pl.ds(0, _CPT), pl.ds(0, _TAILW)],
        out_hbm.at[pl.ds(c0, _CPT), pl.ds(_TBASE, _TAILW)], st1).wait()

  return k(planes, gx, gy)


def kernel(input_tensor, grid):
  planes = jnp.pad(input_tensor[0], ((0, 0), (1, 2), (1, 3)))
  planes = planes.reshape(_C, _PL)
  g = grid.reshape(_P, 2)
  gx = jnp.pad(g[:, 0], (0, _P_PAD - _P))
  gy = jnp.pad(g[:, 1], (0, _P_PAD - _P))
  out = _sc_grid_sample(planes, gx, gy)
  return out[:, :_P].reshape(1, _C, _HG, _WG)


# speedup vs baseline: 2.1423x; 1.1429x over previous
import functools

import jax
import jax.numpy as jnp
from jax import lax
from jax.experimental import pallas as pl
from jax.experimental.pallas import tpu as pltpu
from jax.experimental.pallas import tpu_sc as plsc

_C = 128
_H = 128
_W = 128
_HG = 7
_WG = 25281
_P = _HG * _WG
_PR = _H + 3
_PSTR = _W + 4
_PL = _PR * _PSTR
_B = 4096
_NFULL = _P // _B
_TAILW = 1024
_TBASE = _NFULL * _B
_P_PAD = _TBASE + _TAILW
_NPAIR = 21
_NTILE = 32
_CPT = _C // _NTILE
_NG = _B // 16
_UNROLL = 6


def _sc_grid_sample(planes, gx, gy):
  mesh = plsc.VectorSubcoreMesh(core_axis_name="c", subcore_axis_name="s")

  @functools.partial(
      pl.kernel,
      out_type=jax.ShapeDtypeStruct((_C, _P_PAD), jnp.float32),
      mesh=mesh,
      compiler_params=pltpu.CompilerParams(needs_layout_passes=False),
      scratch_types=[
          pltpu.VMEM((_PL,), jnp.float32),
          pltpu.VMEM((_PL,), jnp.float32),
          pltpu.VMEM((_PL,), jnp.float32),
          pltpu.VMEM((_PL,), jnp.float32),
          pltpu.VMEM((_B,), jnp.float32),
          pltpu.VMEM((_B,), jnp.float32),
          pltpu.VMEM((_B,), jnp.float32),
          pltpu.VMEM((_B,), jnp.float32),
          pltpu.VMEM((_CPT, _B), jnp.float32),
          pltpu.VMEM((_CPT, _B), jnp.float32),
          pltpu.SemaphoreType.DMA,
          pltpu.SemaphoreType.DMA,
          pltpu.SemaphoreType.DMA,
          pltpu.SemaphoreType.DMA,
      ],
  )
  def k(planes_hbm, gx_hbm, gy_hbm, out_hbm, p0, p1, p2, p3,
        gx0, gy0, gx1, gy1, out0, out1, in0, in1, st0, st1):
    wid = lax.axis_index("c") * 16 + lax.axis_index("s")
    c0 = wid * _CPT
    pltpu.sync_copy(planes_hbm.at[c0], p0)
    pltpu.sync_copy(planes_hbm.at[c0 + 1], p1)
    pltpu.sync_copy(planes_hbm.at[c0 + 2], p2)
    pltpu.sync_copy(planes_hbm.at[c0 + 3], p3)

    def start_in(base, gxb, gyb, sem):
      pltpu.async_copy(gx_hbm.at[pl.ds(base, _B)], gxb, sem)
      pltpu.async_copy(gy_hbm.at[pl.ds(base, _B)], gyb, sem)

    def wait_in(gxb, gyb, sem):
      pltpu.make_async_copy(gx_hbm.at[pl.ds(0, _B)], gxb, sem).wait()
      pltpu.make_async_copy(gy_hbm.at[pl.ds(0, _B)], gyb, sem).wait()

    def start_store(base, outb, sem):
      pltpu.async_copy(outb, out_hbm.at[pl.ds(c0, _CPT), pl.ds(base, _B)],
                       sem)

    def wait_store(outb, sem):
      pltpu.make_async_copy(
          outb, out_hbm.at[pl.ds(c0, _CPT), pl.ds(0, _B)], sem).wait()

    def compute(gxb, gyb, outb, ngroups):
      @plsc.parallel_loop(0, ngroups, step=1, unroll=_UNROLL)
      def group_body(g):
        off = g * 16
        gx16 = gxb[pl.ds(off, 16)]
        gy16 = gyb[pl.ds(off, 16)]
        ix = ((gx16 + 1.0) * 128.0 - 1.0) * 0.5
        iy = ((gy16 + 1.0) * 128.0 - 1.0) * 0.5
        ix = jnp.minimum(jnp.maximum(ix, -1.0), 128.0)
        iy = jnp.minimum(jnp.maximum(iy, -1.0), 128.0)
        bx = ix + 1.0
        by = iy + 1.0
        px0 = bx.astype(jnp.int32)
        py0 = by.astype(jnp.int32)
        wx1 = bx - px0.astype(jnp.float32)
        wy1 = by - py0.astype(jnp.float32)
        wx0 = 1.0 - wx1
        wy0 = 1.0 - wy1
        rb = (py0 << 7) + (py0 << 2) + px0
        i01 = rb + 1
        i10 = rb + _PSTR
        i11 = rb + (_PSTR + 1)
        w00 = wx0 * wy0
        w01 = wx1 * wy0
        w10 = wx0 * wy1
        w11 = wx1 * wy1
        for c, pv in enumerate((p0, p1, p2, p3)):
          v00 = plsc.load_gather(pv, [rb])
          v01 = plsc.load_gather(pv, [i01])
          v10 = plsc.load_gather(pv, [i10])
          v11 = plsc.load_gather(pv, [i11])
          acc = v00 * w00 + v01 * w01 + v10 * w10 + v11 * w11
          outb[c, pl.ds(off, 16)] = acc

    start_in(0, gx0, gy0, in0)

    def pair_body(i, carry):
      start_in((2 * i + 1) * _B, gx1, gy1, in1)
      wait_in(gx0, gy0, in0)

      @pl.when(i > 0)
      def _():
        wait_store(out0, st0)

      compute(gx0, gy0, out0, _NG)
      start_store(2 * i * _B, out0, st0)
      start_in((2 * i + 2) * _B, gx0, gy0, in0)
      wait_in(gx1, gy1, in1)

      @pl.when(i > 0)
      def _():
        wait_store(out1, st1)

      compute(gx1, gy1, out1, _NG)
      start_store((2 * i + 1) * _B, out1, st1)
      return carry

    lax.fori_loop(0, _NPAIR, pair_body, 0)

    pltpu.async_copy(gx_hbm.at[pl.ds(_TBASE, _TAILW)],
                     gx1.at[pl.ds(0, _TAILW)], in1)
    pltpu.async_copy(gy_hbm.at[pl.ds(_TBASE, _TAILW)],
                     gy1.at[pl.ds(0, _TAILW)], in1)
    wait_in(gx0, gy0, in0)
    wait_store(out0, st0)
    compute(gx0, gy0, out0, _NG)
    start_store(42 * _B, out0, st0)
    pltpu.make_async_copy(gx_hbm.at[pl.ds(0, _TAILW)],
                          gx1.at[pl.ds(0, _TAILW)], in1).wait()
    pltpu.make_async_copy(gy_hbm.at[pl.ds(0, _TAILW)],
                          gy1.at[pl.ds(0, _TAILW)], in1).wait()
    wait_store(out1, st1)
    compute(gx1, gy1, out1, _TAILW // 16)
    pltpu.async_copy(out1.at[pl.ds(0, _CPT), pl.ds(0, _TAILW)],
                     out_hbm.at[pl.ds(c0, _CPT), pl.ds(_TBASE, _TAILW)],
                     st1)
    wait_store(out0, st0)
    pltpu.make_async_copy(
        out1.at[pl.ds(0, _CPT), pl.ds(0, _TAILW)],
        out_hbm.at[pl.ds(c0, _CPT), pl.ds(_TBASE, _TAILW)], st1).wait()

  return k(planes, gx, gy)


def kernel(input_tensor, grid):
  planes = jnp.pad(input_tensor[0], ((0, 0), (1, 2), (1, 3)))
  planes = planes.reshape(_C, _PL)
  g = grid.reshape(_P, 2)
  gx = jnp.pad(g[:, 0], (0, _P_PAD - _P))
  gy = jnp.pad(g[:, 1], (0, _P_PAD - _P))
  out = _sc_grid_sample(planes, gx, gy)
  return out[:, :_P].reshape(1, _C, _HG, _WG)
